# baseline (speedup 1.0000x reference)
import functools

import jax
import jax.numpy as jnp
from jax import lax
from jax.experimental import pallas as pl
from jax.experimental.pallas import tpu as pltpu
from jax.experimental.pallas import tpu_sc as plsc

NX, NY = 432, 496
NCELL = NX * NY
B = 2
P = 4096
N = 4096
C = 64
K = 4
CB = 128
PB = 512
NB = P // PB
CHUNK = 512
SPATF = CB * NCELL
PINDF = 3 * NCELL


def _topk_idx(S, n_rows, n_cols):
    iota = lax.broadcasted_iota(jnp.int32, (n_rows, n_cols), 0)
    idxs = []
    for _ in range(K):
        m = jnp.max(S, axis=0)
        idx = jnp.min(jnp.where(S == m[None, :], iota, n_rows), axis=0)
        idxs.append(idx)
        S = jnp.where(iota == idx[None, :], -jnp.inf, S)
    return idxs


def _k1_body(points_ref, pillars_ref, coords_ref, pcoords_ref, a_ref,
             z1_ref, z2_ref):
    pts = points_ref[0]
    pil = pillars_ref[0]
    crd = coords_ref[0]
    pcd = pcoords_ref[0]
    amat = a_ref[...]

    f32 = jnp.float32
    dn = (((1,), (1,)), ((), ()))
    S1 = lax.dot_general(pts, pil, dn, preferred_element_type=f32)
    idx1 = _topk_idx(S1, N, PB)

    S2 = lax.dot_general(pcd, crd, dn, preferred_element_type=f32)
    m2 = jnp.max(S2, axis=0, keepdims=True)
    u2 = jnp.exp(S2 - m2)
    s2 = u2 / jnp.sum(u2, axis=0, keepdims=True)
    idx2 = _topk_idx(s2, N, PB)

    Y = lax.dot_general(pts, amat, (((1,), (0,)), ((), ())),
                        preferred_element_type=f32)

    iota_n = lax.broadcasted_iota(jnp.int32, (PB, N), 1)
    z1 = jnp.zeros((PB, C), f32)
    z2 = jnp.zeros((PB, C), f32)
    for k in range(K):
        oh1 = jnp.where(idx1[k][:, None] == iota_n, 1.0, 0.0).astype(f32)
        oh2 = jnp.where(idx2[k][:, None] == iota_n, 1.0, 0.0).astype(f32)
        yk = Y[:, k * C:(k + 1) * C]
        z1 = z1 + lax.dot_general(oh1, yk, (((1,), (0,)), ((), ())),
                                  preferred_element_type=f32)
        z2 = z2 + lax.dot_general(oh2, yk, (((1,), (0,)), ((), ())),
                                  preferred_element_type=f32)
    z1_ref[0] = z1
    z2_ref[0] = z2


def _run_k1(points, pillars, coords, pcoords, amat):
    grid = (B, NB)
    return pl.pallas_call(
        _k1_body,
        grid=grid,
        in_specs=[
            pl.BlockSpec((1, N, C), lambda b, j: (b, 0, 0)),
            pl.BlockSpec((1, PB, C), lambda b, j: (b, j, 0)),
            pl.BlockSpec((1, PB, 4), lambda b, j: (b, j, 0)),
            pl.BlockSpec((1, N, 4), lambda b, j: (b, 0, 0)),
            pl.BlockSpec((C, K * C), lambda b, j: (0, 0)),
        ],
        out_specs=[
            pl.BlockSpec((1, PB, C), lambda b, j: (b, j, 0)),
            pl.BlockSpec((1, PB, C), lambda b, j: (b, j, 0)),
        ],
        out_shape=[
            jax.ShapeDtypeStruct((B, P, C), jnp.float32),
            jax.ShapeDtypeStruct((B, P, C), jnp.float32),
        ],
    )(points, pillars, coords, pcoords, amat)


def _k2_body(z1_ref, z2_ref, pillars_ref, coords_ref, ww_ref, g2_ref,
             b2_ref, g1_ref, b1_ref, full_cm_ref, lin_ref, pxy_ref):
    f32 = jnp.float32
    z1 = z1_ref[0]
    z2 = z2_ref[0]
    pil = pillars_ref[0]
    crd = coords_ref[0]
    ww = ww_ref[...]
    g2 = g2_ref[...]
    b2 = b2_ref[...]
    g1 = g1_ref[...]
    b1 = b1_ref[...]

    u = lax.dot_general(pil, ww, (((1,), (1,)), ((), ())),
                        preferred_element_type=f32)
    mu_u = jnp.mean(u, axis=0, keepdims=True)
    var_u = jnp.mean((u - mu_u) ** 2, axis=0, keepdims=True)
    ubn = (u - mu_u) / jnp.sqrt(var_u + 1e-3) * g2 + b2
    mrow = jnp.max(ubn, axis=1, keepdims=True)
    eu = jnp.exp(ubn - mrow)
    w = eu / jnp.sum(eu, axis=1, keepdims=True)

    z = w[:, 0:1] * z1 + w[:, 1:2] * z2
    mu = jnp.mean(z, axis=0, keepdims=True)
    var = jnp.mean((z - mu) ** 2, axis=0, keepdims=True)
    pp = jnp.maximum((z - mu) / jnp.sqrt(var + 1e-3) * g1 + b1, 0.0)

    fullT = jnp.concatenate([pil, pp], axis=1)

    lin_f = crd[:, 1] + crd[:, 2] * NX + crd[:, 3]
    lin = lin_f.astype(jnp.int32)
    lin_ref[0, 0] = lin
    pxy_ref[0, 0] = crd[:, 2]
    pxy_ref[0, 1] = crd[:, 3]

    iota_q = lax.broadcasted_iota(jnp.int32, (CHUNK, P), 1)
    for ch in range(P // CHUNK):
        linc = lin[ch * CHUNK:(ch + 1) * CHUNK]
        eq = linc[:, None] == lin[None, :]
        wsel = jnp.max(jnp.where(eq, iota_q, -1), axis=1)
        oh = jnp.where(iota_q == wsel[:, None], 1.0, 0.0).astype(f32)
        resT = lax.dot_general(fullT, oh, (((0,), (1,)), ((), ())),
                               preferred_element_type=f32)
        full_cm_ref[0, :, ch * CHUNK:(ch + 1) * CHUNK] = resT


def _run_k2(z1, z2, pillars, coords, ww, g2, b2, g1, b1):
    grid = (B,)
    return pl.pallas_call(
        _k2_body,
        grid=grid,
        in_specs=[
            pl.BlockSpec((1, P, C), lambda b: (b, 0, 0)),
            pl.BlockSpec((1, P, C), lambda b: (b, 0, 0)),
            pl.BlockSpec((1, P, C), lambda b: (b, 0, 0)),
            pl.BlockSpec((1, P, 4), lambda b: (b, 0, 0)),
            pl.BlockSpec((2, C), lambda b: (0, 0)),
            pl.BlockSpec((1, 2), lambda b: (0, 0)),
            pl.BlockSpec((1, 2), lambda b: (0, 0)),
            pl.BlockSpec((1, C), lambda b: (0, 0)),
            pl.BlockSpec((1, C), lambda b: (0, 0)),
        ],
        out_specs=[
            pl.BlockSpec((1, CB, P), lambda b: (b, 0, 0)),
            pl.BlockSpec((1, 1, P), lambda b: (b, 0, 0)),
            pl.BlockSpec((1, 2, P), lambda b: (b, 0, 0)),
        ],
        out_shape=[
            jax.ShapeDtypeStruct((B, CB, P), jnp.float32),
            jax.ShapeDtypeStruct((B, 1, P), jnp.int32),
            jax.ShapeDtypeStruct((B, 2, P), jnp.float32),
        ],
    )(z1, z2, pillars, coords, ww, g2, b2, g1, b1)


TPP = P // 16
ZSP = SPATF // 16
ZPI = PINDF // 16
ZBUF = 16384
ZSP_N = ZSP // ZBUF
ZSP_T = ZSP - ZSP_N * ZBUF
ZPI_N = ZPI // ZBUF
ZPI_T = ZPI - ZPI_N * ZBUF


def _sc_body(full_cm, linB, pxyT, spat_out, pind_out,
             zbuf, linv, slab, idxb, pxys, pidx, zsem, ssem):
    b = lax.axis_index("c")
    t = lax.axis_index("s")

    def zb(i, _):
        zbuf[pl.ds(i * 16, 16)] = jnp.zeros((16,), jnp.float32)
        return 0
    lax.fori_loop(0, ZBUF // 16, zb, 0)

    sbase = b * SPATF + t * ZSP
    pbase = b * PINDF + t * ZPI

    def zfire(i, _):
        pltpu.async_copy(zbuf, spat_out.at[pl.ds(sbase + i * ZBUF, ZBUF)],
                         zsem)
        return 0
    lax.fori_loop(0, ZSP_N, zfire, 0)
    pltpu.async_copy(zbuf.at[pl.ds(0, ZSP_T)],
                     spat_out.at[pl.ds(sbase + ZSP_N * ZBUF, ZSP_T)], zsem)
    for i in range(ZPI_N):
        pltpu.async_copy(zbuf, pind_out.at[pl.ds(pbase + i * ZBUF, ZBUF)],
                         zsem)
    pltpu.async_copy(zbuf.at[pl.ds(0, ZPI_T)],
                     pind_out.at[pl.ds(pbase + ZPI_N * ZBUF, ZPI_T)], zsem)

    def zdrain(i, _):
        pltpu.make_async_copy(
            zbuf, spat_out.at[pl.ds(sbase + i * ZBUF, ZBUF)], zsem).wait()
        return 0
    lax.fori_loop(0, ZSP_N, zdrain, 0)
    pltpu.make_async_copy(
        zbuf.at[pl.ds(0, ZSP_T)],
        spat_out.at[pl.ds(sbase + ZSP_N * ZBUF, ZSP_T)], zsem).wait()
    for i in range(ZPI_N):
        pltpu.make_async_copy(
            zbuf, pind_out.at[pl.ds(pbase + i * ZBUF, ZBUF)], zsem).wait()
    pltpu.make_async_copy(
        zbuf.at[pl.ds(0, ZPI_T)],
        pind_out.at[pl.ds(pbase + ZPI_N * ZBUF, ZPI_T)], zsem).wait()

    plsc.subcore_barrier()

    p0 = t * TPP
    pltpu.sync_copy(linB.at[b, 0, pl.ds(p0, TPP)], linv)
    for j in range(2):
        pltpu.sync_copy(full_cm.at[b, :, pl.ds(p0 + j * 128, 128)],
                        slab.at[j])
    pltpu.sync_copy(pxyT.at[b, :, pl.ds(p0, TPP)], pxys)

    def ibuild(c, _):
        for j in range(2):
            for v in range(8):
                lv = linv[pl.ds(j * 128 + v * 16, 16)]
                idxb[j, c, pl.ds(v * 16, 16)] = lv + (c * NCELL + b * SPATF)
        return 0
    lax.fori_loop(0, CB, ibuild, 0)
    for plane in range(2):
        for j in range(2):
            for v in range(8):
                lv = linv[pl.ds(j * 128 + v * 16, 16)]
                pidx[plane * 2 + j, pl.ds(v * 16, 16)] = (
                    lv + (plane * NCELL + b * PINDF))

    def sfire(i, _):
        for u in range(8):
            c = i * 8 + u
            for j in range(2):
                pltpu.async_copy(slab.at[j, c],
                                 spat_out.at[idxb.at[j, c]], ssem)
        for u in range(8):
            c = i * 8 + u
            for j in range(2):
                pltpu.make_async_copy(
                    slab.at[j, c],
                    spat_out.at[idxb.at[j, c]], ssem).wait()
        return 0
    lax.fori_loop(0, CB // 8, sfire, 0)

    for plane in range(2):
        for j in range(2):
            pltpu.async_copy(pxys.at[plane, pl.ds(j * 128, 128)],
                             pind_out.at[pidx.at[plane * 2 + j]], ssem)
    for plane in range(2):
        for j in range(2):
            pltpu.make_async_copy(
                pxys.at[plane, pl.ds(j * 128, 128)],
                pind_out.at[pidx.at[plane * 2 + j]], ssem).wait()


def _run_k3(full_cm, linB, pxyT):
    mesh = plsc.VectorSubcoreMesh(core_axis_name="c", subcore_axis_name="s")
    f = pl.kernel(
        _sc_body,
        mesh=mesh,
        out_type=[
            jax.ShapeDtypeStruct((B * SPATF,), jnp.float32),
            jax.ShapeDtypeStruct((B * PINDF,), jnp.float32),
        ],
        scratch_types=[
            pltpu.VMEM((ZBUF,), jnp.float32),
            pltpu.VMEM((TPP,), jnp.int32),
            pltpu.VMEM((2, 128, 128), jnp.float32),
            pltpu.VMEM((2, CB, 128), jnp.int32),
            pltpu.VMEM((2, TPP), jnp.float32),
            pltpu.VMEM((4, 128), jnp.int32),
            pltpu.SemaphoreType.DMA,
            pltpu.SemaphoreType.DMA,
        ],
    )
    return f(full_cm, linB, pxyT)


def kernel(pillar_features, voxel_coords, pillar_mask, point_features,
           point_coords, W_adapt, gamma1, beta1, W_weight, gamma2, beta2):
    del pillar_mask
    pillars = pillar_features.reshape(B, P, C)
    coords = voxel_coords.reshape(B, P, 4)
    points = point_features.reshape(B, N, C)
    pcoords = point_coords.reshape(B, N, 4)
    amat = W_adapt.reshape(C, K, C).transpose(2, 1, 0).reshape(C, K * C)

    z1, z2 = _run_k1(points, pillars, coords, pcoords, amat)
    full_cm, lin3, pxyT = _run_k2(
        z1, z2, pillars, coords, W_weight,
        gamma2.reshape(1, 2), beta2.reshape(1, 2),
        gamma1.reshape(1, C), beta1.reshape(1, C))
    spat, pind = _run_k3(full_cm, lin3, pxyT)
    return (spat.reshape(B, CB, NY, NX), pind.reshape(B, 3, NY, NX))
